# speedup vs baseline: 2.9022x; 2.9022x over previous
import functools

import jax
import jax.numpy as jnp
import numpy as np
from jax.experimental import pallas as pl
from jax.experimental.pallas import tpu as pltpu

HIDDEN = 128
CELL = 10
MAX_NF = 3
X_DIM = MAX_NF + CELL
N_NODES = 10000
N_EDGES = 160000
COMBOS = ((0, 0), (0, 1), (1, 1))

EDGE_BLK = 2000
NODE_BLK = 1000

_DOT = functools.partial(jnp.dot, precision=jax.lax.Precision.HIGHEST)


def _ln(h, g, b):
    m = jnp.mean(h, axis=-1, keepdims=True)
    v = jnp.mean((h - m) * (h - m), axis=-1, keepdims=True)
    return (h - m) * jax.lax.rsqrt(v + 1e-5) * g + b


def _node_body(inp_ref, w1_ref, b1_ref, g1_ref, be1_ref, w2_ref, b2_ref,
               g2_ref, be2_ref, out_ref):
    inp = inp_ref[...]
    t = inp[:, 3:4]
    acc = None
    for i in range(2):
        h = _DOT(inp, w1_ref[i]) + b1_ref[i]
        h = jax.nn.relu(_ln(h, g1_ref[i], be1_ref[i]))
        h = _DOT(h, w2_ref[i]) + b2_ref[i]
        h = jnp.tanh(_ln(h, g2_ref[i], be2_ref[i]))
        acc = h if acc is None else jnp.where(t == 0.0, acc, h)
    out_ref[...] = acc


def _edge_body(inp_ref, w1_ref, b1_ref, g1_ref, be1_ref, w2_ref, b2_ref,
               g2_ref, be2_ref, out_ref):
    inp = inp_ref[...]
    ts = inp[:, 13:14]
    te = inp[:, 29:30]
    acc = jnp.zeros(out_ref.shape, jnp.float32)
    for j, (a, b) in enumerate(COMBOS):
        h = _DOT(inp, w1_ref[j]) + b1_ref[j]
        h = jax.nn.relu(_ln(h, g1_ref[j], be1_ref[j]))
        h = _DOT(h, w2_ref[j]) + b2_ref[j]
        h = jnp.tanh(_ln(h, g2_ref[j], be2_ref[j]))
        m = (ts == float(a)) & (te == float(b))
        acc = jnp.where(m, h, acc)
    out_ref[...] = acc


def _full(shape):
    return pl.BlockSpec(shape, lambda i: (0,) * len(shape))


def _stack_node_params(node_params):
    w1 = jnp.stack([jnp.pad(p[0][0], ((0, 8 - MAX_NF), (0, 0)))
                    for p in node_params])
    b1 = jnp.stack([p[0][1] for p in node_params])
    g1 = jnp.stack([p[0][2] for p in node_params])
    be1 = jnp.stack([p[0][3] for p in node_params])
    w2 = jnp.stack([p[1][0] for p in node_params])
    b2 = jnp.stack([p[1][1] for p in node_params])
    g2 = jnp.stack([p[1][2] for p in node_params])
    be2 = jnp.stack([p[1][3] for p in node_params])
    return w1, b1, g1, be1, w2, b2, g2, be2


def _stack_edge_params(edge_params):
    w1s = []
    for p in edge_params:
        w = p[0][0]
        wp = jnp.zeros((32, 128), jnp.float32)
        wp = wp.at[0:13].set(w[0:13])
        wp = wp.at[16:29].set(w[13:26])
        w1s.append(wp)
    w1 = jnp.stack(w1s)
    b1 = jnp.stack([p[0][1] for p in edge_params])
    g1 = jnp.stack([p[0][2] for p in edge_params])
    be1 = jnp.stack([p[0][3] for p in edge_params])
    w2 = jnp.stack([p[1][0] for p in edge_params])
    b2 = jnp.stack([p[1][1] for p in edge_params])
    g2 = jnp.stack([p[1][2] for p in edge_params])
    be2 = jnp.stack([p[1][3] for p in edge_params])
    return w1, b1, g1, be1, w2, b2, g2, be2


def kernel(x, edge_index, volume_id, node_params, edge_params):
    t = (volume_id >= 3).astype(jnp.float32)
    xp = jnp.concatenate(
        [x, t[:, None], jnp.zeros((N_NODES, 2), jnp.float32)], axis=1)

    start, end = edge_index[0], edge_index[1]
    einp = jnp.concatenate([xp[start], xp[end]], axis=1)
    ninp = jnp.concatenate([x[:, :MAX_NF], t[:, None],
                            jnp.zeros((N_NODES, 4), jnp.float32)], axis=1)

    nw = _stack_node_params(node_params)
    ew = _stack_edge_params(edge_params)

    wspecs_n = [_full(w.shape) for w in nw]
    encoded_nodes = pl.pallas_call(
        _node_body,
        grid=(N_NODES // NODE_BLK,),
        in_specs=[pl.BlockSpec((NODE_BLK, 8), lambda i: (i, 0))] + wspecs_n,
        out_specs=pl.BlockSpec((NODE_BLK, HIDDEN), lambda i: (i, 0)),
        out_shape=jax.ShapeDtypeStruct((N_NODES, HIDDEN), jnp.float32),
    )(ninp, *nw)

    wspecs_e = [_full(w.shape) for w in ew]
    encoded_edges = pl.pallas_call(
        _edge_body,
        grid=(N_EDGES // EDGE_BLK,),
        in_specs=[pl.BlockSpec((EDGE_BLK, 32), lambda i: (i, 0))] + wspecs_e,
        out_specs=pl.BlockSpec((EDGE_BLK, HIDDEN), lambda i: (i, 0)),
        out_shape=jax.ShapeDtypeStruct((N_EDGES, HIDDEN), jnp.float32),
    )(einp, *ew)

    return (encoded_nodes, encoded_edges)


# speedup vs baseline: 3.6779x; 1.2673x over previous
import functools

import jax
import jax.numpy as jnp
import numpy as np
from jax.experimental import pallas as pl
from jax.experimental.pallas import tpu as pltpu

HIDDEN = 128
CELL = 10
MAX_NF = 3
X_DIM = MAX_NF + CELL
N_NODES = 10000
N_EDGES = 160000
COMBOS = ((0, 0), (0, 1), (1, 1))

EDGE_BLK = 2000
NODE_BLK = 1000

_DOT = functools.partial(jnp.dot, precision=jax.lax.Precision.DEFAULT)


def _ln(h, g, b):
    m = jnp.mean(h, axis=-1, keepdims=True)
    v = jnp.mean((h - m) * (h - m), axis=-1, keepdims=True)
    return (h - m) * jax.lax.rsqrt(v + 1e-5) * g + b


def _node_body(inp_ref, w1_ref, b1_ref, g1_ref, be1_ref, w2_ref, b2_ref,
               g2_ref, be2_ref, out_ref):
    inp = inp_ref[...]
    t = inp[:, 3:4]
    acc = None
    for i in range(2):
        h = _DOT(inp, w1_ref[i]) + b1_ref[i]
        h = jax.nn.relu(_ln(h, g1_ref[i], be1_ref[i]))
        h = _DOT(h, w2_ref[i]) + b2_ref[i]
        h = jnp.tanh(_ln(h, g2_ref[i], be2_ref[i]))
        acc = h if acc is None else jnp.where(t == 0.0, acc, h)
    out_ref[...] = acc


def _edge_body(inp_ref, w1_ref, b1_ref, g1_ref, be1_ref, w2_ref, b2_ref,
               g2_ref, be2_ref, out_ref):
    inp = inp_ref[...]
    ts = inp[:, 13:14]
    te = inp[:, 29:30]
    acc = jnp.zeros(out_ref.shape, jnp.float32)
    for j, (a, b) in enumerate(COMBOS):
        h = _DOT(inp, w1_ref[j]) + b1_ref[j]
        h = jax.nn.relu(_ln(h, g1_ref[j], be1_ref[j]))
        h = _DOT(h, w2_ref[j]) + b2_ref[j]
        h = jnp.tanh(_ln(h, g2_ref[j], be2_ref[j]))
        m = (ts == float(a)) & (te == float(b))
        acc = jnp.where(m, h, acc)
    out_ref[...] = acc


def _full(shape):
    return pl.BlockSpec(shape, lambda i: (0,) * len(shape))


def _stack_node_params(node_params):
    w1 = jnp.stack([jnp.pad(p[0][0], ((0, 8 - MAX_NF), (0, 0)))
                    for p in node_params])
    b1 = jnp.stack([p[0][1] for p in node_params])
    g1 = jnp.stack([p[0][2] for p in node_params])
    be1 = jnp.stack([p[0][3] for p in node_params])
    w2 = jnp.stack([p[1][0] for p in node_params])
    b2 = jnp.stack([p[1][1] for p in node_params])
    g2 = jnp.stack([p[1][2] for p in node_params])
    be2 = jnp.stack([p[1][3] for p in node_params])
    return w1, b1, g1, be1, w2, b2, g2, be2


def _stack_edge_params(edge_params):
    w1s = []
    for p in edge_params:
        w = p[0][0]
        wp = jnp.zeros((32, 128), jnp.float32)
        wp = wp.at[0:13].set(w[0:13])
        wp = wp.at[16:29].set(w[13:26])
        w1s.append(wp)
    w1 = jnp.stack(w1s)
    b1 = jnp.stack([p[0][1] for p in edge_params])
    g1 = jnp.stack([p[0][2] for p in edge_params])
    be1 = jnp.stack([p[0][3] for p in edge_params])
    w2 = jnp.stack([p[1][0] for p in edge_params])
    b2 = jnp.stack([p[1][1] for p in edge_params])
    g2 = jnp.stack([p[1][2] for p in edge_params])
    be2 = jnp.stack([p[1][3] for p in edge_params])
    return w1, b1, g1, be1, w2, b2, g2, be2


def kernel(x, edge_index, volume_id, node_params, edge_params):
    t = (volume_id >= 3).astype(jnp.float32)
    xp = jnp.concatenate(
        [x, t[:, None], jnp.zeros((N_NODES, 2), jnp.float32)], axis=1)

    start, end = edge_index[0], edge_index[1]
    einp = jnp.concatenate([xp[start], xp[end]], axis=1)
    ninp = jnp.concatenate([x[:, :MAX_NF], t[:, None],
                            jnp.zeros((N_NODES, 4), jnp.float32)], axis=1)

    nw = _stack_node_params(node_params)
    ew = _stack_edge_params(edge_params)

    wspecs_n = [_full(w.shape) for w in nw]
    encoded_nodes = pl.pallas_call(
        _node_body,
        grid=(N_NODES // NODE_BLK,),
        in_specs=[pl.BlockSpec((NODE_BLK, 8), lambda i: (i, 0))] + wspecs_n,
        out_specs=pl.BlockSpec((NODE_BLK, HIDDEN), lambda i: (i, 0)),
        out_shape=jax.ShapeDtypeStruct((N_NODES, HIDDEN), jnp.float32),
    )(ninp, *nw)

    wspecs_e = [_full(w.shape) for w in ew]
    encoded_edges = pl.pallas_call(
        _edge_body,
        grid=(N_EDGES // EDGE_BLK,),
        in_specs=[pl.BlockSpec((EDGE_BLK, 32), lambda i: (i, 0))] + wspecs_e,
        out_specs=pl.BlockSpec((EDGE_BLK, HIDDEN), lambda i: (i, 0)),
        out_shape=jax.ShapeDtypeStruct((N_EDGES, HIDDEN), jnp.float32),
    )(einp, *ew)

    return (encoded_nodes, encoded_edges)


# speedup vs baseline: 7.4924x; 2.0372x over previous
import functools

import jax
import jax.numpy as jnp
from jax import lax
from jax.experimental import pallas as pl
from jax.experimental.pallas import tpu as pltpu
from jax.experimental.pallas import tpu_sc as plsc

HIDDEN = 128
MAX_NF = 3
N_NODES = 10000
N_EDGES = 160000
COMBOS = ((0, 0), (0, 1), (1, 1))

EDGE_BLK = 2000
NODE_BLK = 1000

NWORKERS = 32
E_PAD = 163840
CHUNK = E_PAD // NWORKERS
PIECE = 128
NPIECE = CHUNK // PIECE

_DOT = functools.partial(jnp.dot, precision=jax.lax.Precision.DEFAULT)


def _ln(h, g, b):
    m = jnp.mean(h, axis=-1, keepdims=True)
    v = jnp.mean((h - m) * (h - m), axis=-1, keepdims=True)
    return (h - m) * lax.rsqrt(v + 1e-5) * g + b


def _gather_body(sidx_hbm, didx_hbm, xp_hbm, src_out, dst_out,
                 sidx_v, didx_v, srows_v, drows_v, sem_s, sem_d):
    wid = lax.axis_index("s") * 2 + lax.axis_index("c")
    base = wid * CHUNK
    pltpu.sync_copy(sidx_hbm.at[pl.ds(base, CHUNK)], sidx_v)
    pltpu.sync_copy(didx_hbm.at[pl.ds(base, CHUNK)], didx_v)

    def piece(i, carry):
        off = i * PIECE
        cs = pltpu.async_copy(
            xp_hbm.at[sidx_v.at[pl.ds(off, PIECE)]], srows_v, sem_s)
        cd = pltpu.async_copy(
            xp_hbm.at[didx_v.at[pl.ds(off, PIECE)]], drows_v, sem_d)
        cs.wait()
        cd.wait()
        pltpu.sync_copy(srows_v, src_out.at[pl.ds(base + off, PIECE)])
        pltpu.sync_copy(drows_v, dst_out.at[pl.ds(base + off, PIECE)])
        return carry

    lax.fori_loop(0, NPIECE, piece, 0)


def _sc_gather(start_pad, end_pad, xp):
    mesh = plsc.VectorSubcoreMesh(core_axis_name="c", subcore_axis_name="s")
    f = pl.kernel(
        _gather_body,
        out_type=(
            jax.ShapeDtypeStruct((E_PAD, 16), jnp.float32),
            jax.ShapeDtypeStruct((E_PAD, 16), jnp.float32),
        ),
        mesh=mesh,
        compiler_params=pltpu.CompilerParams(use_tc_tiling_on_sc=False),
        scratch_types=[
            pltpu.VMEM((CHUNK,), jnp.int32),
            pltpu.VMEM((CHUNK,), jnp.int32),
            pltpu.VMEM((PIECE, 16), jnp.float32),
            pltpu.VMEM((PIECE, 16), jnp.float32),
            pltpu.SemaphoreType.DMA,
            pltpu.SemaphoreType.DMA,
        ],
    )
    return f(start_pad, end_pad, xp)


def _node_body(inp_ref, w1_ref, b1_ref, g1_ref, be1_ref, w2_ref, b2_ref,
               g2_ref, be2_ref, out_ref):
    inp = inp_ref[...]
    t = inp[:, 3:4]
    acc = None
    for i in range(2):
        h = _DOT(inp, w1_ref[i]) + b1_ref[i]
        h = jax.nn.relu(_ln(h, g1_ref[i], be1_ref[i]))
        h = _DOT(h, w2_ref[i]) + b2_ref[i]
        h = jnp.tanh(_ln(h, g2_ref[i], be2_ref[i]))
        acc = h if acc is None else jnp.where(t == 0.0, acc, h)
    out_ref[...] = acc


def _edge_body(src_ref, dst_ref, w1a_ref, w1b_ref, b1_ref, g1_ref, be1_ref,
               w2_ref, b2_ref, g2_ref, be2_ref, out_ref):
    src = src_ref[...]
    dst = dst_ref[...]
    ts = src[:, 13:14]
    te = dst[:, 13:14]
    acc = jnp.zeros(out_ref.shape, jnp.float32)
    for j, (a, b) in enumerate(COMBOS):
        h = _DOT(src, w1a_ref[j]) + _DOT(dst, w1b_ref[j]) + b1_ref[j]
        h = jax.nn.relu(_ln(h, g1_ref[j], be1_ref[j]))
        h = _DOT(h, w2_ref[j]) + b2_ref[j]
        h = jnp.tanh(_ln(h, g2_ref[j], be2_ref[j]))
        m = (ts == float(a)) & (te == float(b))
        acc = jnp.where(m, h, acc)
    out_ref[...] = acc


def _full(shape):
    return pl.BlockSpec(shape, lambda i: (0,) * len(shape))


def _stack_node_params(node_params):
    w1 = jnp.stack([jnp.pad(p[0][0], ((0, 8 - MAX_NF), (0, 0)))
                    for p in node_params])
    rest = [jnp.stack([p[li][ai] for p in node_params])
            for li in (0, 1) for ai in (1, 2, 3)]
    w2 = jnp.stack([p[1][0] for p in node_params])
    b1, g1, be1, b2, g2, be2 = rest
    return w1, b1, g1, be1, w2, b2, g2, be2


def _stack_edge_params(edge_params):
    w1a = jnp.stack([jnp.pad(p[0][0][0:13], ((0, 3), (0, 0)))
                     for p in edge_params])
    w1b = jnp.stack([jnp.pad(p[0][0][13:26], ((0, 3), (0, 0)))
                     for p in edge_params])
    rest = [jnp.stack([p[li][ai] for p in edge_params])
            for li in (0, 1) for ai in (1, 2, 3)]
    w2 = jnp.stack([p[1][0] for p in edge_params])
    b1, g1, be1, b2, g2, be2 = rest
    return w1a, w1b, b1, g1, be1, w2, b2, g2, be2


def kernel(x, edge_index, volume_id, node_params, edge_params):
    t = (volume_id >= 3).astype(jnp.float32)
    xp = jnp.concatenate(
        [x, t[:, None], jnp.zeros((N_NODES, 2), jnp.float32)], axis=1)

    start_pad = jnp.pad(edge_index[0], (0, E_PAD - N_EDGES))
    end_pad = jnp.pad(edge_index[1], (0, E_PAD - N_EDGES))
    src, dst = _sc_gather(start_pad, end_pad, xp)

    ninp = jnp.concatenate([x[:, :MAX_NF], t[:, None],
                            jnp.zeros((N_NODES, 4), jnp.float32)], axis=1)

    nw = _stack_node_params(node_params)
    encoded_nodes = pl.pallas_call(
        _node_body,
        grid=(N_NODES // NODE_BLK,),
        in_specs=[pl.BlockSpec((NODE_BLK, 8), lambda i: (i, 0))]
        + [_full(w.shape) for w in nw],
        out_specs=pl.BlockSpec((NODE_BLK, HIDDEN), lambda i: (i, 0)),
        out_shape=jax.ShapeDtypeStruct((N_NODES, HIDDEN), jnp.float32),
    )(ninp, *nw)

    ew = _stack_edge_params(edge_params)
    encoded_edges = pl.pallas_call(
        _edge_body,
        grid=(N_EDGES // EDGE_BLK,),
        in_specs=[pl.BlockSpec((EDGE_BLK, 16), lambda i: (i, 0))] * 2
        + [_full(w.shape) for w in ew],
        out_specs=pl.BlockSpec((EDGE_BLK, HIDDEN), lambda i: (i, 0)),
        out_shape=jax.ShapeDtypeStruct((N_EDGES, HIDDEN), jnp.float32),
    )(src, dst, *ew)

    return (encoded_nodes, encoded_edges)
